# baseline (speedup 1.0000x reference)
import functools

import jax
import jax.numpy as jnp
from jax import lax
from jax.experimental import pallas as pl
from jax.experimental.pallas import tpu as pltpu
from jax.experimental.pallas import tpu_sc as plsc

_B = 16384
_NLINES = 6
_NC, _NS, _L = 2, 16, 16
_NW = _NC * _NS
_ROWS = _B // _NW
_NGRP = _ROWS // _L


def _hex_index_body(lines_hbm, out_hbm, lines_v, idx_v):
    wid = lax.axis_index("s") * _NC + lax.axis_index("c")
    base = wid * _ROWS
    pltpu.sync_copy(
        lines_hbm.at[pl.ds(base * _NLINES, _ROWS * _NLINES)], lines_v)
    lane6 = lax.iota(jnp.int32, 16) * _NLINES

    def group(g, carry):
        word0 = lane6 + g * (_L * _NLINES)
        acc = plsc.load_gather(lines_v, [word0])
        for j in range(1, _NLINES):
            acc = acc + plsc.load_gather(lines_v, [word0 + j]) * float(1 << j)
        idx_v[pl.ds(g * _L, _L)] = acc.astype(jnp.int32)
        return carry

    lax.fori_loop(0, _NGRP, group, 0)
    pltpu.sync_copy(idx_v, out_hbm.at[pl.ds(base, _ROWS)])


_hex_index_sc = functools.partial(
    pl.kernel,
    out_type=jax.ShapeDtypeStruct((_B,), jnp.int32),
    mesh=plsc.VectorSubcoreMesh(
        core_axis_name="c", subcore_axis_name="s",
        num_cores=_NC, num_subcores=_NS),
    scratch_types=[
        pltpu.VMEM((_ROWS * _NLINES,), jnp.float32),
        pltpu.VMEM((_ROWS,), jnp.int32),
    ],
    compiler_params=pltpu.CompilerParams(
        needs_layout_passes=False, skip_device_barrier=True),
)(_hex_index_body)


def kernel(lines, hex_table, line_table):
    hex_index = _hex_index_sc(lines.reshape(_B * _NLINES))
    return (lines, hex_index, lines, jnp.zeros_like(lines))


# speedup vs baseline: 1.0169x; 1.0169x over previous
import functools

import jax
import jax.numpy as jnp
from jax import lax
from jax.experimental import pallas as pl
from jax.experimental.pallas import tpu as pltpu
from jax.experimental.pallas import tpu_sc as plsc

_B = 16384
_NLINES = 6
_NC, _NS, _L = 1, 16, 16
_NW = _NC * _NS
_ROWS = _B // _NW
_NGRP = _ROWS // _L


def _hex_index_body(lines_hbm, out_hbm, lines_v, idx_v):
    wid = lax.axis_index("s") * _NC + lax.axis_index("c")
    base = wid * _ROWS
    pltpu.sync_copy(
        lines_hbm.at[pl.ds(base * _NLINES, _ROWS * _NLINES)], lines_v)
    lane6 = lax.iota(jnp.int32, 16) * _NLINES

    def group(g, carry):
        word0 = lane6 + g * (_L * _NLINES)
        acc = plsc.load_gather(lines_v, [word0])
        for j in range(1, _NLINES):
            acc = acc + plsc.load_gather(lines_v, [word0 + j]) * float(1 << j)
        idx_v[pl.ds(g * _L, _L)] = acc.astype(jnp.int32)
        return carry

    lax.fori_loop(0, _NGRP, group, 0)
    pltpu.sync_copy(idx_v, out_hbm.at[pl.ds(base, _ROWS)])


_hex_index_sc = functools.partial(
    pl.kernel,
    out_type=jax.ShapeDtypeStruct((_B,), jnp.int32),
    mesh=plsc.VectorSubcoreMesh(
        core_axis_name="c", subcore_axis_name="s",
        num_cores=_NC, num_subcores=_NS),
    scratch_types=[
        pltpu.VMEM((_ROWS * _NLINES,), jnp.float32),
        pltpu.VMEM((_ROWS,), jnp.int32),
    ],
    compiler_params=pltpu.CompilerParams(
        needs_layout_passes=False, skip_device_barrier=True),
)(_hex_index_body)


def kernel(lines, hex_table, line_table):
    hex_index = _hex_index_sc(lines.reshape(_B * _NLINES))
    return (lines, hex_index, lines, jnp.zeros_like(lines))
